# Initial kernel scaffold; baseline (speedup 1.0000x reference)
import jax
import jax.numpy as jnp
from jax.experimental import pallas as pl


def kernel(high_dim_features, low_dim_features, edge_index, W_emb, b_emb, W1, b1, W2, b2, W_lin, b_lin):
    raise NotImplementedError("write your pallas kernel here")


# speedup vs baseline: 10.8883x; 10.8883x over previous
import functools

import jax
import jax.numpy as jnp
from jax import lax
from jax.experimental import pallas as pl
from jax.experimental.pallas import tpu as pltpu
from jax.experimental.pallas import tpu_sc as plsc

N = 10000
NPAD = 10240
E = 320000
HID = 64
WID = 128
NC = 2
NS = 16
EPW = E // (NC * NS)
CHUNK = 80
NCHUNK = EPW // CHUNK
RPS = NPAD // NS

_MESH = dict(core_axis_name="c", subcore_axis_name="s")


@functools.partial(
    pl.kernel,
    mesh=plsc.VectorSubcoreMesh(**_MESH),
    out_type=jax.ShapeDtypeStruct((2 * NPAD, WID), jnp.float32),
    scratch_types=[
        pltpu.VMEM((CHUNK,), jnp.int32),
        pltpu.VMEM((CHUNK, WID), jnp.float32),
        pltpu.VMEM((64, WID), jnp.float32),
        pltpu.VMEM_SHARED((NPAD, WID), jnp.float32),
    ],
)
def _sc_degree(dst_hbm, out_hbm, idx_v, ones_v, zbuf_v, hist_s):
    c = lax.axis_index("c")
    s = lax.axis_index("s")
    zero16 = jnp.zeros((16,), jnp.float32)
    one16 = jnp.ones((16,), jnp.float32)
    for r in range(64):
        for j in range(WID // 16):
            zbuf_v[r, pl.ds(j * 16, 16)] = zero16
    for r in range(CHUNK):
        for j in range(WID // 16):
            ones_v[r, pl.ds(j * 16, 16)] = one16
    for k in range(RPS // 64):
        pltpu.sync_copy(zbuf_v, hist_s.at[pl.ds(s * RPS + k * 64, 64)])
    plsc.subcore_barrier()
    base = (c * NS + s) * EPW

    def body(k, carry):
        off = pl.multiple_of(base + k * CHUNK, CHUNK)
        pltpu.sync_copy(dst_hbm.at[pl.ds(off, CHUNK)], idx_v)
        pltpu.sync_copy(ones_v, hist_s.at[idx_v], add=True)
        return carry

    lax.fori_loop(0, NCHUNK, body, 0)
    plsc.subcore_barrier()
    dst_off = pl.multiple_of(c * NPAD + s * RPS, RPS)
    pltpu.sync_copy(hist_s.at[pl.ds(s * RPS, RPS)],
                    out_hbm.at[pl.ds(dst_off, RPS)])


@functools.partial(
    pl.kernel,
    mesh=plsc.VectorSubcoreMesh(**_MESH),
    out_type=jax.ShapeDtypeStruct((2 * NPAD, WID), jnp.float32),
    scratch_types=[
        pltpu.VMEM((CHUNK,), jnp.int32),
        pltpu.VMEM((CHUNK,), jnp.int32),
        pltpu.VMEM((CHUNK, WID), jnp.float32),
        pltpu.VMEM((64, WID), jnp.float32),
        pltpu.VMEM_SHARED((NPAD, WID), jnp.float32),
        pltpu.SemaphoreType.DMA,
    ],
)
def _sc_agg(h_hbm, src_hbm, dst_hbm, out_hbm,
            sidx_v, didx_v, rows_v, zbuf_v, acc_s, sem):
    c = lax.axis_index("c")
    s = lax.axis_index("s")
    zero16 = jnp.zeros((16,), jnp.float32)
    for r in range(64):
        for j in range(WID // 16):
            zbuf_v[r, pl.ds(j * 16, 16)] = zero16
    for k in range(RPS // 64):
        pltpu.sync_copy(zbuf_v, acc_s.at[pl.ds(s * RPS + k * 64, 64)])
    plsc.subcore_barrier()
    base = (c * NS + s) * EPW

    def body(k, carry):
        off = pl.multiple_of(base + k * CHUNK, CHUNK)
        pltpu.sync_copy(src_hbm.at[pl.ds(off, CHUNK)], sidx_v)
        pltpu.sync_copy(dst_hbm.at[pl.ds(off, CHUNK)], didx_v)
        pltpu.async_copy(h_hbm.at[sidx_v], rows_v, sem).wait()
        pltpu.sync_copy(rows_v, acc_s.at[didx_v], add=True)
        return carry

    lax.fori_loop(0, NCHUNK, body, 0)
    plsc.subcore_barrier()
    dst_off = pl.multiple_of(c * NPAD + s * RPS, RPS)
    pltpu.sync_copy(acc_s.at[pl.ds(s * RPS, RPS)],
                    out_hbm.at[pl.ds(dst_off, RPS)])


_R = 1000
_G = N // _R


def _rows(cols):
    return pl.BlockSpec((_R, cols), lambda i: (i, 0))


def _full(shape):
    return pl.BlockSpec(shape, lambda i: (0,) * len(shape))


def _dinv(d0_ref, d1_ref):
    deg = d0_ref[:, 0:1] + d1_ref[:, 0:1] + 1.0
    return lax.rsqrt(deg)


def _tc_embed_body(high_ref, low_ref, wemb_ref, bemb_ref, w1a_ref, w1b_ref,
                   out_ref):
    le = jnp.maximum(low_ref[...] @ wemb_ref[...] + bemb_ref[...], 0.0)
    out_ref[...] = high_ref[...] @ w1a_ref[...] + le @ w1b_ref[...]


def _pad_wid(x):
    return jnp.concatenate([x, jnp.zeros_like(x)], axis=1)


def _tc_scale_body(d0_ref, d1_ref, h_ref, out_ref, dv_ref):
    dinv = _dinv(d0_ref, d1_ref)
    out_ref[...] = _pad_wid(h_ref[...] * dinv)
    dv_ref[...] = jnp.broadcast_to(dinv, dv_ref.shape)


def _tc_conv_mid_body(p0_ref, p1_ref, hp_ref, dv_ref, b1_ref, w2_ref,
                      out_ref):
    dinv = dv_ref[:, 0:1]
    msg = (p0_ref[...] + p1_ref[...] + hp_ref[...])[:, :HID]
    x = jnp.maximum(dinv * msg + b1_ref[...], 0.0)
    out_ref[...] = _pad_wid(dinv * (x @ w2_ref[...]))


def _tc_head_body(q0_ref, q1_ref, hp_ref, dv_ref, b2_ref, wl_ref,
                  bl_ref, out_ref):
    dinv = dv_ref[:, 0:1]
    msg = (q0_ref[...] + q1_ref[...] + hp_ref[...])[:, :HID]
    x = jnp.maximum(dinv * msg + b2_ref[...], 0.0)
    logits = x @ wl_ref[...] + bl_ref[...]
    m = jnp.max(logits, axis=1, keepdims=True)
    lse = m + jnp.log(jnp.sum(jnp.exp(logits - m), axis=1, keepdims=True))
    out_ref[...] = logits - lse


def _tc_embed(high, low, wemb, bemb, w1a, w1b):
    return pl.pallas_call(
        _tc_embed_body,
        grid=(_G,),
        in_specs=[_rows(128), _rows(16), _full((16, HID)), _full((1, HID)),
                  _full((128, HID)), _full((HID, HID))],
        out_specs=_rows(HID),
        out_shape=jax.ShapeDtypeStruct((N, HID), jnp.float32),
    )(high, low, wemb, bemb, w1a, w1b)


def _tc_scale(d0, d1, h):
    return pl.pallas_call(
        _tc_scale_body,
        grid=(_G,),
        in_specs=[_rows(WID), _rows(WID), _rows(HID)],
        out_specs=(_rows(WID), _rows(16)),
        out_shape=(jax.ShapeDtypeStruct((N, WID), jnp.float32),
                   jax.ShapeDtypeStruct((N, 16), jnp.float32)),
    )(d0, d1, h)


def _tc_conv_mid(p0, p1, hp, dv, b1, w2):
    return pl.pallas_call(
        _tc_conv_mid_body,
        grid=(_G,),
        in_specs=[_rows(WID), _rows(WID), _rows(WID), _rows(16),
                  _full((1, HID)), _full((HID, HID))],
        out_specs=_rows(WID),
        out_shape=jax.ShapeDtypeStruct((N, WID), jnp.float32),
    )(p0, p1, hp, dv, b1, w2)


def _tc_head(q0, q1, hp, dv, b2, wl, bl):
    return pl.pallas_call(
        _tc_head_body,
        grid=(_G,),
        in_specs=[_rows(WID), _rows(WID), _rows(WID), _rows(16),
                  _full((1, HID)), _full((HID, 16)), _full((1, 16))],
        out_specs=_rows(16),
        out_shape=jax.ShapeDtypeStruct((N, 16), jnp.float32),
    )(q0, q1, hp, dv, b2, wl, bl)


def kernel(high_dim_features, low_dim_features, edge_index, W_emb, b_emb,
           W1, b1, W2, b2, W_lin, b_lin):
    ei = edge_index.astype(jnp.int32)
    src = ei[0]
    dst = ei[1]

    degp = _sc_degree(dst)
    d0 = degp[:N]
    d1 = degp[NPAD:NPAD + N]

    h1 = _tc_embed(high_dim_features, low_dim_features, W_emb,
                   b_emb.reshape(1, -1), W1[:128], W1[128:])
    h1p, dv = _tc_scale(d0, d1, h1)

    p = _sc_agg(h1p, src, dst)
    h2p = _tc_conv_mid(p[:N], p[NPAD:NPAD + N], h1p, dv,
                       b1.reshape(1, -1), W2)

    q = _sc_agg(h2p, src, dst)
    return _tc_head(q[:N], q[NPAD:NPAD + N], h2p, dv,
                    b2.reshape(1, -1), W_lin, b_lin.reshape(1, -1))


# speedup vs baseline: 22.6455x; 2.0798x over previous
import functools

import jax
import jax.numpy as jnp
from jax import lax
from jax.experimental import pallas as pl
from jax.experimental.pallas import tpu as pltpu
from jax.experimental.pallas import tpu_sc as plsc

N = 10000
NPAD = 10240
E = 320000
HID = 64
WID = 128
NC = 2
NS = 16
EPW = E // (NC * NS)
CHUNK = 80
NCHUNK = EPW // CHUNK
RPS = NPAD // NS

_MESH = dict(core_axis_name="c", subcore_axis_name="s")


def _fill_idx(dbuf, all_idx, ch):
    for j in range(CHUNK // 16):
        dbuf[pl.ds(j * 16, 16)] = all_idx[pl.ds(ch * CHUNK + j * 16, 16)]


@functools.partial(
    pl.kernel,
    mesh=plsc.VectorSubcoreMesh(**_MESH),
    out_type=jax.ShapeDtypeStruct((2 * NPAD, WID), jnp.float32),
    scratch_types=[
        pltpu.VMEM((EPW,), jnp.int32),
        pltpu.VMEM((CHUNK,), jnp.int32),
        pltpu.VMEM((CHUNK,), jnp.int32),
        pltpu.VMEM((CHUNK, WID), jnp.float32),
        pltpu.VMEM((64, WID), jnp.float32),
        pltpu.VMEM_SHARED((NPAD, WID), jnp.float32),
        pltpu.SemaphoreType.DMA,
        pltpu.SemaphoreType.DMA,
    ],
)
def _sc_degree(dst_hbm, out_hbm, didx_all, db0, db1, ones_v, zbuf_v, hist_s,
               sem0, sem1):
    c = lax.axis_index("c")
    s = lax.axis_index("s")
    zero16 = jnp.zeros((16,), jnp.float32)
    one16 = jnp.ones((16,), jnp.float32)
    for r in range(64):
        for j in range(WID // 16):
            zbuf_v[r, pl.ds(j * 16, 16)] = zero16
    for r in range(CHUNK):
        for j in range(WID // 16):
            ones_v[r, pl.ds(j * 16, 16)] = one16
    base = pl.multiple_of(((c * NS + s) * EPW).astype(jnp.int32), CHUNK)
    pltpu.sync_copy(dst_hbm.at[pl.ds(base, EPW)], didx_all)
    for k in range(RPS // 64):
        pltpu.sync_copy(zbuf_v, hist_s.at[pl.ds(s * RPS + k * 64, 64)])
    plsc.subcore_barrier()

    _fill_idx(db0, didx_all, 0)

    def body(i, carry):
        d0 = pltpu.async_copy(ones_v, hist_s.at[db0], sem0, add=True)
        _fill_idx(db1, didx_all, 2 * i + 1)
        d1 = pltpu.async_copy(ones_v, hist_s.at[db1], sem1, add=True)
        d0.wait()
        _fill_idx(db0, didx_all, 2 * i + 2)
        d1.wait()
        return carry

    lax.fori_loop(0, (NCHUNK - 1) // 2, body, 0)
    pltpu.sync_copy(ones_v, hist_s.at[db0], add=True)
    plsc.subcore_barrier()
    dst_off = pl.multiple_of(c * NPAD + s * RPS, RPS)
    pltpu.sync_copy(hist_s.at[pl.ds(s * RPS, RPS)],
                    out_hbm.at[pl.ds(dst_off, RPS)])


@functools.partial(
    pl.kernel,
    mesh=plsc.VectorSubcoreMesh(**_MESH),
    out_type=jax.ShapeDtypeStruct((2 * NPAD, WID), jnp.float32),
    scratch_types=[
        pltpu.VMEM((EPW,), jnp.int32),
        pltpu.VMEM((EPW,), jnp.int32),
        pltpu.VMEM((CHUNK,), jnp.int32),
        pltpu.VMEM((CHUNK,), jnp.int32),
        pltpu.VMEM((CHUNK, WID), jnp.float32),
        pltpu.VMEM((CHUNK, WID), jnp.float32),
        pltpu.VMEM((64, WID), jnp.float32),
        pltpu.VMEM_SHARED((NPAD, WID), jnp.float32),
        pltpu.SemaphoreType.DMA,
        pltpu.SemaphoreType.DMA,
    ],
)
def _sc_agg(h_hbm, src_hbm, dst_hbm, out_hbm,
            sidx_all, didx_all, db0, db1, rows0, rows1, zbuf_v, acc_s,
            gsem0, gsem1):
    c = lax.axis_index("c")
    s = lax.axis_index("s")
    zero16 = jnp.zeros((16,), jnp.float32)
    for r in range(64):
        for j in range(WID // 16):
            zbuf_v[r, pl.ds(j * 16, 16)] = zero16
    base = pl.multiple_of(((c * NS + s) * EPW).astype(jnp.int32), CHUNK)
    pltpu.sync_copy(src_hbm.at[pl.ds(base, EPW)], sidx_all)
    pltpu.sync_copy(dst_hbm.at[pl.ds(base, EPW)], didx_all)
    for k in range(RPS // 64):
        pltpu.sync_copy(zbuf_v, acc_s.at[pl.ds(s * RPS + k * 64, 64)])
    plsc.subcore_barrier()

    def gather(ch, rows, sem):
        return pltpu.async_copy(
            h_hbm.at[sidx_all.at[pl.ds(ch * CHUNK, CHUNK)]], rows, sem)

    _fill_idx(db0, didx_all, 0)
    gather(0, rows0, gsem0)

    def body(i, carry):
        ch0 = 2 * i
        _fill_idx(db1, didx_all, ch0 + 1)
        gather(ch0 + 1, rows1, gsem1)
        pltpu.make_async_copy(
            h_hbm.at[sidx_all.at[pl.ds(ch0 * CHUNK, CHUNK)]], rows0,
            gsem0).wait()
        pltpu.sync_copy(rows0, acc_s.at[db0], add=True)
        _fill_idx(db0, didx_all, ch0 + 2)
        gather(ch0 + 2, rows0, gsem0)
        pltpu.make_async_copy(
            h_hbm.at[sidx_all.at[pl.ds((ch0 + 1) * CHUNK, CHUNK)]], rows1,
            gsem1).wait()
        pltpu.sync_copy(rows1, acc_s.at[db1], add=True)
        return carry

    lax.fori_loop(0, (NCHUNK - 1) // 2, body, 0)
    pltpu.make_async_copy(
        h_hbm.at[sidx_all.at[pl.ds((NCHUNK - 1) * CHUNK, CHUNK)]], rows0,
        gsem0).wait()
    pltpu.sync_copy(rows0, acc_s.at[db0], add=True)
    plsc.subcore_barrier()
    dst_off = pl.multiple_of(c * NPAD + s * RPS, RPS)
    pltpu.sync_copy(acc_s.at[pl.ds(s * RPS, RPS)],
                    out_hbm.at[pl.ds(dst_off, RPS)])


_R = 1000
_G = N // _R


def _rows(cols):
    return pl.BlockSpec((_R, cols), lambda i: (i, 0))


def _full(shape):
    return pl.BlockSpec(shape, lambda i: (0,) * len(shape))


def _dinv(d0_ref, d1_ref):
    deg = d0_ref[:, 0:1] + d1_ref[:, 0:1] + 1.0
    return lax.rsqrt(deg)


def _tc_embed_body(high_ref, low_ref, wemb_ref, bemb_ref, w1a_ref, w1b_ref,
                   out_ref):
    le = jnp.maximum(low_ref[...] @ wemb_ref[...] + bemb_ref[...], 0.0)
    out_ref[...] = high_ref[...] @ w1a_ref[...] + le @ w1b_ref[...]


def _pad_wid(x):
    return jnp.concatenate([x, jnp.zeros_like(x)], axis=1)


def _tc_scale_body(d0_ref, d1_ref, h_ref, out_ref, dv_ref):
    dinv = _dinv(d0_ref, d1_ref)
    out_ref[...] = _pad_wid(h_ref[...] * dinv)
    dv_ref[...] = jnp.broadcast_to(dinv, dv_ref.shape)


def _tc_conv_mid_body(p0_ref, p1_ref, hp_ref, dv_ref, b1_ref, w2_ref,
                      out_ref):
    dinv = dv_ref[:, 0:1]
    msg = (p0_ref[...] + p1_ref[...] + hp_ref[...])[:, :HID]
    x = jnp.maximum(dinv * msg + b1_ref[...], 0.0)
    out_ref[...] = _pad_wid(dinv * (x @ w2_ref[...]))


def _tc_head_body(q0_ref, q1_ref, hp_ref, dv_ref, b2_ref, wl_ref,
                  bl_ref, out_ref):
    dinv = dv_ref[:, 0:1]
    msg = (q0_ref[...] + q1_ref[...] + hp_ref[...])[:, :HID]
    x = jnp.maximum(dinv * msg + b2_ref[...], 0.0)
    logits = x @ wl_ref[...] + bl_ref[...]
    m = jnp.max(logits, axis=1, keepdims=True)
    lse = m + jnp.log(jnp.sum(jnp.exp(logits - m), axis=1, keepdims=True))
    out_ref[...] = logits - lse


def _tc_embed(high, low, wemb, bemb, w1a, w1b):
    return pl.pallas_call(
        _tc_embed_body,
        grid=(_G,),
        in_specs=[_rows(128), _rows(16), _full((16, HID)), _full((1, HID)),
                  _full((128, HID)), _full((HID, HID))],
        out_specs=_rows(HID),
        out_shape=jax.ShapeDtypeStruct((N, HID), jnp.float32),
    )(high, low, wemb, bemb, w1a, w1b)


def _tc_scale(d0, d1, h):
    return pl.pallas_call(
        _tc_scale_body,
        grid=(_G,),
        in_specs=[_rows(WID), _rows(WID), _rows(HID)],
        out_specs=(_rows(WID), _rows(16)),
        out_shape=(jax.ShapeDtypeStruct((N, WID), jnp.float32),
                   jax.ShapeDtypeStruct((N, 16), jnp.float32)),
    )(d0, d1, h)


def _tc_conv_mid(p0, p1, hp, dv, b1, w2):
    return pl.pallas_call(
        _tc_conv_mid_body,
        grid=(_G,),
        in_specs=[_rows(WID), _rows(WID), _rows(WID), _rows(16),
                  _full((1, HID)), _full((HID, HID))],
        out_specs=_rows(WID),
        out_shape=jax.ShapeDtypeStruct((N, WID), jnp.float32),
    )(p0, p1, hp, dv, b1, w2)


def _tc_head(q0, q1, hp, dv, b2, wl, bl):
    return pl.pallas_call(
        _tc_head_body,
        grid=(_G,),
        in_specs=[_rows(WID), _rows(WID), _rows(WID), _rows(16),
                  _full((1, HID)), _full((HID, 16)), _full((1, 16))],
        out_specs=_rows(16),
        out_shape=jax.ShapeDtypeStruct((N, 16), jnp.float32),
    )(q0, q1, hp, dv, b2, wl, bl)


def kernel(high_dim_features, low_dim_features, edge_index, W_emb, b_emb,
           W1, b1, W2, b2, W_lin, b_lin):
    ei = edge_index.astype(jnp.int32)
    src = ei[0]
    dst = ei[1]

    degp = _sc_degree(dst)
    d0 = degp[:N]
    d1 = degp[NPAD:NPAD + N]

    h1 = _tc_embed(high_dim_features, low_dim_features, W_emb,
                   b_emb.reshape(1, -1), W1[:128], W1[128:])
    h1p, dv = _tc_scale(d0, d1, h1)

    p = _sc_agg(h1p, src, dst)
    h2p = _tc_conv_mid(p[:N], p[NPAD:NPAD + N], h1p, dv,
                       b1.reshape(1, -1), W2)

    q = _sc_agg(h2p, src, dst)
    return _tc_head(q[:N], q[NPAD:NPAD + N], h2p, dv,
                    b2.reshape(1, -1), W_lin, b_lin.reshape(1, -1))
